# baseline (speedup 1.0000x reference)
import functools

import jax
import jax.numpy as jnp
from jax import lax
from jax.experimental import pallas as pl
from jax.experimental.pallas import tpu as pltpu
from jax.experimental.pallas import tpu_sc as plsc

VOCAB = 1000000
SEQ = 200
DIM = 32
BATCH = 4096
LANES = 16

NC = 2
NS = 16
NW = NC * NS
BW = BATCH // NW
GB = 8
TP = 129
NBUF = 4
LOOK = 3


def _body(idxT_hbm, word_hbm, posT_hbm, out_hbm, idx_v, pos_v, g_v, t_v,
          gsem, ssem):
    w = lax.axis_index("s") * NC + lax.axis_index("c")
    pltpu.sync_copy(idxT_hbm.at[:, pl.ds(w * BW, BW)], idx_v)
    pltpu.sync_copy(posT_hbm, pos_v)

    def issue_gather(s, slot):
        pltpu.async_copy(word_hbm.at[idx_v.at[s]], g_v.at[slot],
                         gsem.at[slot])

    for k in range(LOOK):
        issue_gather(k, k)

    row_lo = jnp.arange(LANES, dtype=jnp.int32)
    row_hi = row_lo + LANES

    def it(s, carry):
        slot = lax.rem(s, NBUF)

        pltpu.make_async_copy(
            word_hbm.at[pl.ds(0, BW)], g_v.at[slot], gsem.at[slot]).wait()

        @pl.when(s + LOOK < SEQ)
        def _():
            issue_gather(s + LOOK, lax.rem(s + LOOK, NBUF))

        @pl.when(s >= NBUF)
        def _():
            for dI in range(DIM // 8):
                pltpu.make_async_copy(
                    t_v.at[slot, pl.ds(dI * 8, 8), pl.ds(0, 128)],
                    out_hbm.at[0, 0, 0], ssem.at[slot]).wait()

        scol = jnp.zeros((LANES,), jnp.int32) + s
        pv_lo = plsc.load_gather(pos_v, [row_lo, scol])
        pv_hi = plsc.load_gather(pos_v, [row_hi, scol])

        for b0 in range(0, BW, GB):
            vs = []
            for b in range(b0, b0 + GB):
                vs.append((g_v[slot, b, pl.ds(0, LANES)],
                           g_v[slot, b, pl.ds(LANES, LANES)]))
            for i, b in enumerate(range(b0, b0 + GB)):
                col = jnp.full((LANES,), b, jnp.int32)
                plsc.store_scatter(t_v.at[slot], [row_lo, col],
                                   vs[i][0] + pv_lo)
                plsc.store_scatter(t_v.at[slot], [row_hi, col],
                                   vs[i][1] + pv_hi)
            del vs

        for dI in range(DIM // 8):
            pltpu.async_copy(
                t_v.at[slot, pl.ds(dI * 8, 8), pl.ds(0, 128)],
                out_hbm.at[s, dI, w], ssem.at[slot])
        return carry

    lax.fori_loop(0, SEQ, it, 0)

    for t in range(NBUF):
        for dI in range(DIM // 8):
            pltpu.make_async_copy(
                t_v.at[t, pl.ds(dI * 8, 8), pl.ds(0, 128)],
                out_hbm.at[0, 0, 0], ssem.at[t]).wait()


_grid_kernel = functools.partial(
    pl.kernel,
    mesh=plsc.VectorSubcoreMesh(core_axis_name="c", subcore_axis_name="s"),
    out_type=jax.ShapeDtypeStruct(
        (SEQ, DIM // 8, BATCH // 128, 8, 128), jnp.float32),
    compiler_params=pltpu.CompilerParams(
        use_tc_tiling_on_sc=False, needs_layout_passes=False),
    scratch_types=[
        pltpu.VMEM((SEQ, BW), jnp.int32),
        pltpu.VMEM((DIM, SEQ), jnp.float32),
        pltpu.VMEM((NBUF, BW, 128), jnp.float32),
        pltpu.VMEM((NBUF, DIM, TP), jnp.float32),
        pltpu.SemaphoreType.DMA((NBUF,)),
        pltpu.SemaphoreType.DMA((NBUF,)),
    ],
)(_body)


def kernel(inputs, word_table, pos_table):
    wpad = jnp.pad(word_table, ((0, 0), (0, 128 - DIM)))
    out5 = _grid_kernel(inputs.T, wpad, pos_table.T)
    return out5.transpose(2, 4, 0, 1, 3).reshape(BATCH, SEQ, DIM)


# speedup vs baseline: 1.1362x; 1.1362x over previous
import functools

import jax
import jax.numpy as jnp
from jax import lax
from jax.experimental import pallas as pl
from jax.experimental.pallas import tpu as pltpu
from jax.experimental.pallas import tpu_sc as plsc

VOCAB = 1000000
SEQ = 200
DIM = 32
BATCH = 4096
LANES = 16

NC = 2
NS = 16
NW = NC * NS
BW = BATCH // NW
GB = 8
TP = 129
NBUF = 4
LOOK = 3


def _body(idxT_hbm, word_hbm, posT_hbm, out_hbm, idx_v, pos_v, g_v, t_v,
          gsem, ssem):
    w = lax.axis_index("s") * NC + lax.axis_index("c")
    pltpu.sync_copy(idxT_hbm.at[:, pl.ds(w * BW, BW)], idx_v)
    pltpu.sync_copy(posT_hbm, pos_v)

    def issue_gather(s, slot):
        pltpu.async_copy(word_hbm.at[idx_v.at[s]], g_v.at[slot],
                         gsem.at[slot])

    for k in range(LOOK):
        issue_gather(k, k)

    row_lo = jnp.arange(LANES, dtype=jnp.int32)
    row_hi = row_lo + LANES

    def it(s, carry):
        slot = lax.rem(s, NBUF)

        pltpu.make_async_copy(
            word_hbm.at[pl.ds(0, BW)], g_v.at[slot], gsem.at[slot]).wait()

        @pl.when(s + LOOK < SEQ)
        def _():
            issue_gather(s + LOOK, lax.rem(s + LOOK, NBUF))

        @pl.when(s >= NBUF)
        def _():
            for dI in range(DIM // 8):
                pltpu.make_async_copy(
                    t_v.at[slot, pl.ds(dI * 8, 8), pl.ds(0, 128)],
                    out_hbm.at[0, 0, 0], ssem.at[slot]).wait()

        scol = jnp.zeros((LANES,), jnp.int32) + s
        pv_lo = plsc.load_gather(pos_v, [row_lo, scol])
        pv_hi = plsc.load_gather(pos_v, [row_hi, scol])

        for b0 in range(0, BW, GB):
            vs = []
            for b in range(b0, b0 + GB):
                vs.append((g_v[slot, b, pl.ds(0, LANES)],
                           g_v[slot, b, pl.ds(LANES, LANES)]))
            for i, b in enumerate(range(b0, b0 + GB)):
                col = jnp.full((LANES,), b, jnp.int32)
                plsc.store_scatter(t_v.at[slot], [row_lo, col],
                                   vs[i][0] + pv_lo)
                plsc.store_scatter(t_v.at[slot], [row_hi, col],
                                   vs[i][1] + pv_hi)
            del vs

        for dI in range(DIM // 8):
            pltpu.async_copy(
                t_v.at[slot, pl.ds(dI * 8, 8), pl.ds(0, 128)],
                out_hbm.at[s, dI, w], ssem.at[slot])
        return carry

    lax.fori_loop(0, SEQ, it, 0)

    for t in range(NBUF):
        for dI in range(DIM // 8):
            pltpu.make_async_copy(
                t_v.at[t, pl.ds(dI * 8, 8), pl.ds(0, 128)],
                out_hbm.at[0, 0, 0], ssem.at[t]).wait()


_grid_kernel = functools.partial(
    pl.kernel,
    mesh=plsc.VectorSubcoreMesh(core_axis_name="c", subcore_axis_name="s"),
    out_type=jax.ShapeDtypeStruct(
        (SEQ, DIM // 8, BATCH // 128, 8, 128), jnp.float32),
    compiler_params=pltpu.CompilerParams(
        use_tc_tiling_on_sc=False, needs_layout_passes=False),
    scratch_types=[
        pltpu.VMEM((SEQ, BW), jnp.int32),
        pltpu.VMEM((DIM, SEQ), jnp.float32),
        pltpu.VMEM((NBUF, BW, DIM), jnp.float32),
        pltpu.VMEM((NBUF, DIM, TP), jnp.float32),
        pltpu.SemaphoreType.DMA((NBUF,)),
        pltpu.SemaphoreType.DMA((NBUF,)),
    ],
)(_body)


def kernel(inputs, word_table, pos_table):
    out5 = _grid_kernel(inputs.T, word_table, pos_table.T)
    return out5.transpose(2, 4, 0, 1, 3).reshape(BATCH, SEQ, DIM)
